# baseline (speedup 1.0000x reference)
import functools

import jax
import jax.numpy as jnp
from jax import lax
from jax.experimental import pallas as pl
from jax.experimental.pallas import tpu as pltpu
from jax.experimental.pallas import tpu_sc as plsc

GRAPH_NUM = 1000000
EMB_DIM = 64
OUT_DIM = 64
BATCH = 16384

NUM_CORES = 2
NUM_SUBCORES = 16
NW = NUM_CORES * NUM_SUBCORES
B_PER_W = BATCH // NW
IDX_CHUNK = 128
N_CHUNKS = B_PER_W // IDX_CHUNK


def _make_sc_gather():
    mesh = plsc.VectorSubcoreMesh(core_axis_name="c", subcore_axis_name="s")

    @functools.partial(
        pl.kernel,
        mesh=mesh,
        compiler_params=pltpu.CompilerParams(use_tc_tiling_on_sc=False),
        out_type=jax.ShapeDtypeStruct((BATCH, EMB_DIM), jnp.float32),
        scratch_types=[
            pltpu.VMEM((N_CHUNKS, IDX_CHUNK), jnp.int32),
            pltpu.VMEM((B_PER_W, EMB_DIM), jnp.float32),
            pltpu.SemaphoreType.DMA,
        ],
    )
    def gather(table_hbm, idx_hbm, out_hbm, idx_v, rows_v, sem):
        wid = lax.axis_index("s") * NUM_CORES + lax.axis_index("c")
        base = wid * B_PER_W
        pltpu.sync_copy(idx_hbm.at[pl.ds(wid * N_CHUNKS, N_CHUNKS)], idx_v)
        copies = []
        for j in range(N_CHUNKS):
            copies.append(
                pltpu.async_copy(
                    table_hbm.at[idx_v.at[j]],
                    rows_v.at[pl.ds(j * IDX_CHUNK, IDX_CHUNK)],
                    sem,
                )
            )
        for c in copies:
            c.wait()
        pltpu.sync_copy(rows_v, out_hbm.at[pl.ds(base, B_PER_W)])

    return gather


_sc_gather = _make_sc_gather()


def _proj_body(emb_ref, w_ref, out_ref):
    e = emb_ref[...]
    y = lax.dot_general(
        e, w_ref[...],
        dimension_numbers=(((1,), (1,)), ((), ())),
        preferred_element_type=jnp.float32,
    )
    norm = jnp.sqrt(jnp.sum(y * y, axis=-1, keepdims=True))
    out_ref[...] = y / jnp.maximum(norm, 1e-12)


_PROJ_BLOCK = 2048


def _tc_project(emb, W):
    grid = (BATCH // _PROJ_BLOCK,)
    return pl.pallas_call(
        _proj_body,
        grid=grid,
        in_specs=[
            pl.BlockSpec((_PROJ_BLOCK, EMB_DIM), lambda i: (i, 0)),
            pl.BlockSpec((OUT_DIM, EMB_DIM), lambda i: (0, 0)),
        ],
        out_specs=pl.BlockSpec((_PROJ_BLOCK, OUT_DIM), lambda i: (i, 0)),
        out_shape=jax.ShapeDtypeStruct((BATCH, OUT_DIM), jnp.float32),
    )(emb, W)


def kernel(graph_id_list, table, W):
    idx = graph_id_list.astype(jnp.int32).reshape(BATCH // IDX_CHUNK, IDX_CHUNK)
    emb = _sc_gather(table, idx)
    return _tc_project(emb, W)


# speedup vs baseline: 2.3512x; 2.3512x over previous
import functools

import jax
import jax.numpy as jnp
from jax import lax
from jax.experimental import pallas as pl
from jax.experimental.pallas import tpu as pltpu
from jax.experimental.pallas import tpu_sc as plsc

GRAPH_NUM = 1000000
EMB_DIM = 64
OUT_DIM = 64
BATCH = 16384

NUM_CORES = 2
NUM_SUBCORES = 16
NW = NUM_CORES * NUM_SUBCORES
B_PER_W = BATCH // NW
GRP = 16
N_GRP = B_PER_W // GRP
NBUF = 4


def _make_sc_gather():
    mesh = plsc.VectorSubcoreMesh(core_axis_name="c", subcore_axis_name="s")

    @functools.partial(
        pl.kernel,
        mesh=mesh,
        compiler_params=pltpu.CompilerParams(
            use_tc_tiling_on_sc=True, needs_layout_passes=False),
        out_type=jax.ShapeDtypeStruct((BATCH, EMB_DIM), jnp.float32),
        scratch_types=[
            pltpu.VMEM((B_PER_W,), jnp.int32),
            pltpu.VMEM((B_PER_W, EMB_DIM), jnp.float32),
        ]
        + [pltpu.VMEM((EMB_DIM, 128), jnp.float32) for _ in range(NBUF)]
        + [pltpu.SemaphoreType.DMA for _ in range(NBUF)],
    )
    def gather(tT_hbm, idx_hbm, out_hbm, idx_v, out_v, *bufsems):
        bufs = bufsems[:NBUF]
        sems = bufsems[NBUF:]
        wid = lax.axis_index("s") * NUM_CORES + lax.axis_index("c")
        base = wid * B_PER_W
        pltpu.sync_copy(idx_hbm.at[pl.ds(base, B_PER_W)], idx_v)
        lanes = lax.iota(jnp.int32, GRP)

        @pl.loop(0, N_GRP)
        def _grp(grp):
            vec = idx_v[pl.ds(grp * GRP, GRP)]
            gs = [
                lax.reduce_sum(
                    jnp.where(lanes == j, vec, 0), axes=(0,))
                for j in range(GRP)
            ]

            def fire(i):
                off = pl.multiple_of((gs[i] >> 7) << 7, 128)
                return pltpu.async_copy(
                    tT_hbm.at[:, pl.ds(off, 128)], bufs[i % NBUF],
                    sems[i % NBUF])

            copies = [None] * GRP
            for i in range(NBUF):
                copies[i] = fire(i)
            for i in range(GRP):
                copies[i].wait()
                buf = bufs[i % NBUF]
                col = jnp.full((GRP,), gs[i] & 127, jnp.int32)
                row = grp * GRP + i
                for e in range(EMB_DIM // GRP):
                    vals = plsc.load_gather(buf, [e * GRP + lanes, col])
                    out_v[row, pl.ds(e * GRP, GRP)] = vals
                if i + NBUF < GRP:
                    copies[i + NBUF] = fire(i + NBUF)

        pltpu.sync_copy(out_v, out_hbm.at[pl.ds(base, B_PER_W)])

    return gather


_sc_gather = _make_sc_gather()


def _proj_body(emb_ref, w_ref, outT_ref):
    e = emb_ref[...]
    yT = lax.dot_general(
        w_ref[...], e,
        dimension_numbers=(((1,), (1,)), ((), ())),
        preferred_element_type=jnp.float32,
    )
    norm = jnp.sqrt(jnp.sum(yT * yT, axis=0, keepdims=True))
    outT_ref[...] = yT / jnp.maximum(norm, 1e-12)


_PROJ_BLOCK = 2048


def _tc_project(emb, W):
    grid = (BATCH // _PROJ_BLOCK,)
    return pl.pallas_call(
        _proj_body,
        grid=grid,
        in_specs=[
            pl.BlockSpec((_PROJ_BLOCK, EMB_DIM), lambda i: (i, 0)),
            pl.BlockSpec((OUT_DIM, EMB_DIM), lambda i: (0, 0)),
        ],
        out_specs=pl.BlockSpec((OUT_DIM, _PROJ_BLOCK), lambda i: (0, i)),
        out_shape=jax.ShapeDtypeStruct((OUT_DIM, BATCH), jnp.float32),
    )(emb, W)


def kernel(graph_id_list, table, W):
    idx = graph_id_list.astype(jnp.int32)
    tableT = table.T
    emb = _sc_gather(tableT, idx)
    outT = _tc_project(emb, W)
    return outT.T


# speedup vs baseline: 2.5536x; 1.0861x over previous
import functools

import jax
import jax.numpy as jnp
from jax import lax
from jax.experimental import pallas as pl
from jax.experimental.pallas import tpu as pltpu
from jax.experimental.pallas import tpu_sc as plsc

GRAPH_NUM = 1000000
EMB_DIM = 64
OUT_DIM = 64
BATCH = 16384

NUM_CORES = 2
NUM_SUBCORES = 16
NW = NUM_CORES * NUM_SUBCORES
B_PER_W = BATCH // NW
GRP = 16
N_GRP = B_PER_W // GRP
NBUF = 6


def _make_sc_gather():
    mesh = plsc.VectorSubcoreMesh(core_axis_name="c", subcore_axis_name="s")

    @functools.partial(
        pl.kernel,
        mesh=mesh,
        compiler_params=pltpu.CompilerParams(
            use_tc_tiling_on_sc=True, needs_layout_passes=False),
        out_type=jax.ShapeDtypeStruct((BATCH, EMB_DIM), jnp.float32),
        scratch_types=[
            pltpu.VMEM((B_PER_W,), jnp.int32),
            pltpu.VMEM((B_PER_W, EMB_DIM), jnp.float32),
        ]
        + [pltpu.VMEM((EMB_DIM, 128), jnp.float32) for _ in range(NBUF)]
        + [pltpu.SemaphoreType.DMA for _ in range(NBUF)],
    )
    def gather(tT_hbm, idx_hbm, out_hbm, idx_v, out_v, *bufsems):
        bufs = bufsems[:NBUF]
        sems = bufsems[NBUF:]
        wid = lax.axis_index("s") * NUM_CORES + lax.axis_index("c")
        base = wid * B_PER_W
        pltpu.sync_copy(idx_hbm.at[pl.ds(base, B_PER_W)], idx_v)
        lanes = lax.iota(jnp.int32, GRP)

        @pl.loop(0, N_GRP)
        def _grp(grp):
            vec = idx_v[pl.ds(grp * GRP, GRP)]
            gs = [
                lax.reduce_sum(
                    jnp.where(lanes == j, vec, 0), axes=(0,))
                for j in range(GRP)
            ]

            def fire(i):
                off = pl.multiple_of((gs[i] >> 7) << 7, 128)
                return pltpu.async_copy(
                    tT_hbm.at[:, pl.ds(off, 128)], bufs[i % NBUF],
                    sems[i % NBUF])

            copies = [None] * GRP
            for i in range(NBUF):
                copies[i] = fire(i)
            for i in range(GRP):
                copies[i].wait()
                buf = bufs[i % NBUF]
                col = jnp.full((GRP,), gs[i] & 127, jnp.int32)
                row = grp * GRP + i
                for e in range(EMB_DIM // GRP):
                    vals = plsc.load_gather(buf, [e * GRP + lanes, col])
                    out_v[row, pl.ds(e * GRP, GRP)] = vals
                if i + NBUF < GRP:
                    copies[i + NBUF] = fire(i + NBUF)

        pltpu.sync_copy(out_v, out_hbm.at[pl.ds(base, B_PER_W)])

    return gather


_sc_gather = _make_sc_gather()


def _proj_body(emb_ref, w_ref, outT_ref):
    e = emb_ref[...]
    yT = lax.dot_general(
        w_ref[...], e,
        dimension_numbers=(((1,), (1,)), ((), ())),
        preferred_element_type=jnp.float32,
    )
    norm = jnp.sqrt(jnp.sum(yT * yT, axis=0, keepdims=True))
    outT_ref[...] = yT / jnp.maximum(norm, 1e-12)


_PROJ_BLOCK = 2048


def _tc_project(emb, W):
    grid = (BATCH // _PROJ_BLOCK,)
    return pl.pallas_call(
        _proj_body,
        grid=grid,
        in_specs=[
            pl.BlockSpec((_PROJ_BLOCK, EMB_DIM), lambda i: (i, 0)),
            pl.BlockSpec((OUT_DIM, EMB_DIM), lambda i: (0, 0)),
        ],
        out_specs=pl.BlockSpec((OUT_DIM, _PROJ_BLOCK), lambda i: (0, i)),
        out_shape=jax.ShapeDtypeStruct((OUT_DIM, BATCH), jnp.float32),
    )(emb, W)


def kernel(graph_id_list, table, W):
    idx = graph_id_list.astype(jnp.int32)
    tableT = table.T
    emb = _sc_gather(tableT, idx)
    outT = _tc_project(emb, W)
    return outT.T


# speedup vs baseline: 2.5594x; 1.0023x over previous
import functools

import jax
import jax.numpy as jnp
from jax import lax
from jax.experimental import pallas as pl
from jax.experimental.pallas import tpu as pltpu
from jax.experimental.pallas import tpu_sc as plsc

GRAPH_NUM = 1000000
EMB_DIM = 64
OUT_DIM = 64
BATCH = 16384

NUM_CORES = 2
NUM_SUBCORES = 16
NW = NUM_CORES * NUM_SUBCORES
B_PER_W = BATCH // NW
GRP = 16
N_GRP = B_PER_W // GRP
NBUF = 7


def _make_sc_gather():
    mesh = plsc.VectorSubcoreMesh(core_axis_name="c", subcore_axis_name="s")

    @functools.partial(
        pl.kernel,
        mesh=mesh,
        compiler_params=pltpu.CompilerParams(
            use_tc_tiling_on_sc=True, needs_layout_passes=False),
        out_type=jax.ShapeDtypeStruct((BATCH, EMB_DIM), jnp.float32),
        scratch_types=[
            pltpu.VMEM((B_PER_W,), jnp.int32),
            pltpu.VMEM((B_PER_W, EMB_DIM), jnp.float32),
        ]
        + [pltpu.VMEM((EMB_DIM, 128), jnp.float32) for _ in range(NBUF)]
        + [pltpu.SemaphoreType.DMA for _ in range(NBUF)],
    )
    def gather(tT_hbm, idx_hbm, out_hbm, idx_v, out_v, *bufsems):
        bufs = bufsems[:NBUF]
        sems = bufsems[NBUF:]
        wid = lax.axis_index("s") * NUM_CORES + lax.axis_index("c")
        base = wid * B_PER_W
        pltpu.sync_copy(idx_hbm.at[pl.ds(base, B_PER_W)], idx_v)
        lanes = lax.iota(jnp.int32, GRP)

        @pl.loop(0, N_GRP)
        def _grp(grp):
            vec = idx_v[pl.ds(grp * GRP, GRP)]
            gs = [
                lax.reduce_sum(
                    jnp.where(lanes == j, vec, 0), axes=(0,))
                for j in range(GRP)
            ]

            def fire(i):
                off = pl.multiple_of((gs[i] >> 7) << 7, 128)
                return pltpu.async_copy(
                    tT_hbm.at[:, pl.ds(off, 128)], bufs[i % NBUF],
                    sems[i % NBUF])

            copies = [None] * GRP
            for i in range(NBUF):
                copies[i] = fire(i)
            for i in range(GRP):
                copies[i].wait()
                buf = bufs[i % NBUF]
                col = jnp.full((GRP,), gs[i] & 127, jnp.int32)
                row = grp * GRP + i
                for e in range(EMB_DIM // GRP):
                    vals = plsc.load_gather(buf, [e * GRP + lanes, col])
                    out_v[row, pl.ds(e * GRP, GRP)] = vals
                if i + NBUF < GRP:
                    copies[i + NBUF] = fire(i + NBUF)

        pltpu.sync_copy(out_v, out_hbm.at[pl.ds(base, B_PER_W)])

    return gather


_sc_gather = _make_sc_gather()


def _proj_body(emb_ref, w_ref, outT_ref):
    e = emb_ref[...]
    yT = lax.dot_general(
        w_ref[...], e,
        dimension_numbers=(((1,), (1,)), ((), ())),
        preferred_element_type=jnp.float32,
    )
    norm = jnp.sqrt(jnp.sum(yT * yT, axis=0, keepdims=True))
    outT_ref[...] = yT / jnp.maximum(norm, 1e-12)


_PROJ_BLOCK = 2048


def _tc_project(emb, W):
    grid = (BATCH // _PROJ_BLOCK,)
    return pl.pallas_call(
        _proj_body,
        grid=grid,
        in_specs=[
            pl.BlockSpec((_PROJ_BLOCK, EMB_DIM), lambda i: (i, 0)),
            pl.BlockSpec((OUT_DIM, EMB_DIM), lambda i: (0, 0)),
        ],
        out_specs=pl.BlockSpec((OUT_DIM, _PROJ_BLOCK), lambda i: (0, i)),
        out_shape=jax.ShapeDtypeStruct((OUT_DIM, BATCH), jnp.float32),
    )(emb, W)


def kernel(graph_id_list, table, W):
    idx = graph_id_list.astype(jnp.int32)
    tableT = table.T
    emb = _sc_gather(tableT, idx)
    outT = _tc_project(emb, W)
    return outT.T
